# Initial kernel scaffold; baseline (speedup 1.0000x reference)
import jax
import jax.numpy as jnp
from jax.experimental import pallas as pl


def kernel(x, edge_index, edge_attr, Wn, bn_, We, be_, Wf1, bf1, Ws1, bs1, g1, b1, Wf2, bf2, Ws2, bs2, g2, b2, Wl, bl):
    raise NotImplementedError("write your pallas kernel here")


# speedup vs baseline: 1.6844x; 1.6844x over previous
import functools

import jax
import jax.numpy as jnp
from jax import lax
from jax.experimental import pallas as pl
from jax.experimental.pallas import tpu as pltpu
from jax.experimental.pallas import tpu_sc as plsc

N = 50000
E = 800000
H = 64

_info = plsc.get_sparse_core_info()
NC, NS = _info.num_cores, _info.num_subcores
NW = NC * NS
EPW = E // NW
GC = 200


def _gather_body(dst_hbm, src_hbm, t1_hbm, t2_hbm, g1_hbm, g2_hbm, idx_v, rows_v, sem):
    wid = lax.axis_index("s") * NC + lax.axis_index("c")
    base_w = wid * EPW

    def chunk(i, carry):
        base = base_w + i * GC
        pltpu.sync_copy(dst_hbm.at[pl.ds(base, GC)], idx_v)
        pltpu.async_copy(t1_hbm.at[idx_v], rows_v, sem).wait()
        pltpu.sync_copy(rows_v, g1_hbm.at[pl.ds(base, GC)])
        pltpu.sync_copy(src_hbm.at[pl.ds(base, GC)], idx_v)
        pltpu.async_copy(t2_hbm.at[idx_v], rows_v, sem).wait()
        pltpu.sync_copy(rows_v, g2_hbm.at[pl.ds(base, GC)])
        return carry

    lax.fori_loop(0, EPW // GC, chunk, 0)


_sc_gather = functools.partial(
    pl.kernel,
    mesh=plsc.VectorSubcoreMesh(core_axis_name="c", subcore_axis_name="s"),
    out_type=[
        jax.ShapeDtypeStruct((E, 2 * H), jnp.float32),
        jax.ShapeDtypeStruct((E, 2 * H), jnp.float32),
    ],
    scratch_types=[
        pltpu.VMEM((GC,), jnp.int32),
        pltpu.VMEM((GC, 2 * H), jnp.float32),
        pltpu.SemaphoreType.DMA,
    ],
)(_gather_body)


NBLK = 2000


def _proj_body(h_ref, w1_ref, w2_ref, t1_ref, t2_ref):
    h = h_ref[...]
    t1_ref[...] = jnp.dot(h, w1_ref[...], preferred_element_type=jnp.float32)
    t2_ref[...] = jnp.dot(h, w2_ref[...], preferred_element_type=jnp.float32)


def _node_tables(h, Wf, Ws):
    w1 = jnp.concatenate([Wf[0:H], Ws[0:H]], axis=1)
    w2 = jnp.concatenate([Wf[H : 2 * H], Ws[H : 2 * H]], axis=1)
    nb = pl.BlockSpec((NBLK, H), lambda i: (i, 0))
    tb = pl.BlockSpec((NBLK, 2 * H), lambda i: (i, 0))
    wb = pl.BlockSpec((H, 2 * H), lambda i: (0, 0))
    return pl.pallas_call(
        _proj_body,
        grid=(N // NBLK,),
        in_specs=[nb, wb, wb],
        out_specs=[tb, tb],
        out_shape=[
            jax.ShapeDtypeStruct((N, 2 * H), jnp.float32),
            jax.ShapeDtypeStruct((N, 2 * H), jnp.float32),
        ],
    )(h, w1, w2)


MBLK = 2000


def _msg_body(g1_ref, g2_ref, ea_ref, wfe_ref, wse_ref, bf_ref, bs_ref, m_ref):
    g1 = g1_ref[...]
    g2 = g2_ref[...]
    ea = ea_ref[...]
    zf = (
        g1[:, 0:H]
        + g2[:, 0:H]
        + jnp.dot(ea, wfe_ref[...], preferred_element_type=jnp.float32)
        + bf_ref[...]
    )
    zs = (
        g1[:, H : 2 * H]
        + g2[:, H : 2 * H]
        + jnp.dot(ea, wse_ref[...], preferred_element_type=jnp.float32)
        + bs_ref[...]
    )
    softplus = jnp.maximum(zs, 0.0) + jnp.log1p(jnp.exp(-jnp.abs(zs)))
    m_ref[...] = jax.nn.sigmoid(zf) * softplus


def _messages(g1, g2, ea, Wf, bf, Ws, bs):
    gb = pl.BlockSpec((MBLK, 2 * H), lambda i: (i, 0))
    eb = pl.BlockSpec((MBLK, H), lambda i: (i, 0))
    wb = pl.BlockSpec((H, H), lambda i: (0, 0))
    vb = pl.BlockSpec((1, H), lambda i: (0, 0))
    return pl.pallas_call(
        _msg_body,
        grid=(E // MBLK,),
        in_specs=[gb, gb, eb, wb, wb, vb, vb],
        out_specs=eb,
        out_shape=jax.ShapeDtypeStruct((E, H), jnp.float32),
    )(g1, g2, ea, Wf[2 * H :], Ws[2 * H :], bf.reshape(1, H), bs.reshape(1, H))


def _cg_layer(h, dst, src, ea, Wf, bf, Ws, bs, gamma, beta):
    t1, t2 = _node_tables(h, Wf, Ws)
    g1, g2 = _sc_gather(dst, src, t1, t2)
    m = _messages(g1, g2, ea, Wf, bf, Ws, bs)
    agg = jax.ops.segment_max(m, dst, num_segments=N)
    agg = jnp.where(jnp.isneginf(agg), 0.0, agg)
    mu = agg.mean(axis=0)
    var = agg.var(axis=0)
    agg = (agg - mu) / jnp.sqrt(var + 1e-5) * gamma + beta
    return agg + h


def kernel(x, edge_index, edge_attr, Wn, bn_, We, be_, Wf1, bf1, Ws1, bs1, g1, b1, Wf2, bf2, Ws2, bs2, g2, b2, Wl, bl):
    src = edge_index[0]
    dst = edge_index[1]
    h = x @ Wn + bn_
    ea = edge_attr @ We + be_
    h = _cg_layer(h, dst, src, ea, Wf1, bf1, Ws1, bs1, g1, b1)
    h = _cg_layer(h, dst, src, ea, Wf2, bf2, Ws2, bs2, g2, b2)
    logits = h @ Wl + bl
    return (logits, h)
